# baseline (speedup 1.0000x reference)
import functools

import jax
import jax.numpy as jnp
from jax import lax
from jax.experimental import pallas as pl
from jax.experimental.pallas import tpu as pltpu
from jax.experimental.pallas import tpu_sc as plsc

N = 32 * 512 * 512
NC = 2
NS = 16
L = 16
NW = NC * NS
PER_W = N // NW
CHUNK = 16384
NCHUNK = PER_W // CHUNK

_mesh = plsc.VectorSubcoreMesh(core_axis_name="c", subcore_axis_name="s")


@functools.partial(
    pl.kernel,
    mesh=_mesh,
    out_type=jax.ShapeDtypeStruct((NW, 2, L), jnp.float32),
    scratch_types=[
        pltpu.VMEM((2, CHUNK), jnp.float32),
        pltpu.VMEM((2, CHUNK), jnp.float32),
        pltpu.VMEM((2, L), jnp.float32),
        pltpu.SemaphoreType.DMA,
        pltpu.SemaphoreType.DMA,
    ],
)
def _delta1_sc(pred_hbm, target_hbm, out_hbm, pbuf, tbuf, obuf, sem0, sem1):
    cid = lax.axis_index("c")
    sid = lax.axis_index("s")
    wid = sid * NC + cid
    base = wid * PER_W
    sems = (sem0, sem1)

    def start(c):
        slot = c % 2
        dp = pltpu.async_copy(
            pred_hbm.at[pl.ds(base + c * CHUNK, CHUNK)], pbuf.at[slot], sems[slot]
        )
        dt = pltpu.async_copy(
            target_hbm.at[pl.ds(base + c * CHUNK, CHUNK)], tbuf.at[slot], sems[slot]
        )
        return (dp, dt)

    descs = [None] * NCHUNK
    descs[0] = start(0)
    acc = (jnp.zeros((L,), jnp.float32), jnp.zeros((L,), jnp.float32))
    for c in range(NCHUNK):
        if c + 1 < NCHUNK:
            descs[c + 1] = start(c + 1)
        for d in descs[c]:
            d.wait()
        slot = c % 2
        pslot = pbuf.at[slot]
        tslot = tbuf.at[slot]

        def body(i, carry, pslot=pslot, tslot=tslot):
            acc_c, acc_v = carry
            off = pl.multiple_of(i * L, L)
            p = pslot[pl.ds(off, L)]
            t = tslot[pl.ds(off, L)]
            corr = (p < 1.25 * t) & (t < 1.25 * p)
            valid = t > 0.0
            acc_c = acc_c + jnp.where(corr, 1.0, 0.0)
            acc_v = acc_v + jnp.where(valid, 1.0, 0.0)
            return (acc_c, acc_v)

        acc = plsc.parallel_loop(0, CHUNK // L, 1, unroll=8, carry=acc)(body)

    acc_c, acc_v = acc
    obuf.at[0][...] = acc_c
    obuf.at[1][...] = acc_v
    pltpu.sync_copy(obuf, out_hbm.at[wid])


def kernel(pred, target):
    partials = _delta1_sc(pred.reshape(-1), target.reshape(-1))
    sum_c = jnp.sum(partials[:, 0, :])
    sum_v = jnp.sum(partials[:, 1, :])
    acc = sum_c / jnp.maximum(sum_v, 1.0)
    return jnp.where(sum_v < 10.0, jnp.float32(-1.0), acc)


# speedup vs baseline: 1.3839x; 1.3839x over previous
import functools

import jax
import jax.numpy as jnp
from jax import lax
from jax.experimental import pallas as pl
from jax.experimental.pallas import tpu as pltpu
from jax.experimental.pallas import tpu_sc as plsc

N = 32 * 512 * 512
NC = 2
NS = 16
L = 16
NW = NC * NS
PER_W = N // NW
CHUNK = 16384
NCHUNK = PER_W // CHUNK
GPR = 8
ROWS = CHUNK // (GPR * L)

_mesh = plsc.VectorSubcoreMesh(core_axis_name="c", subcore_axis_name="s")


@functools.partial(
    pl.kernel,
    mesh=_mesh,
    out_type=jax.ShapeDtypeStruct((NW, L), jnp.int32),
    scratch_types=[
        pltpu.VMEM((CHUNK,), jnp.float32),
        pltpu.VMEM((CHUNK,), jnp.float32),
        pltpu.VMEM((CHUNK,), jnp.float32),
        pltpu.VMEM((CHUNK,), jnp.float32),
        pltpu.VMEM((L,), jnp.int32),
        pltpu.SemaphoreType.DMA,
        pltpu.SemaphoreType.DMA,
    ],
)
def _delta1_sc(pred_hbm, target_hbm, out_hbm, p0, p1, t0, t1, obuf, sem0, sem1):
    cid = lax.axis_index("c")
    sid = lax.axis_index("s")
    wid = sid * NC + cid
    base = wid * PER_W
    pbufs = (p0, p1)
    tbufs = (t0, t1)
    sems = (sem0, sem1)

    def start(c):
        slot = c % 2
        src = pl.ds(base + c * CHUNK, CHUNK)
        dp = pltpu.async_copy(pred_hbm.at[src], pbufs[slot], sems[slot])
        dt = pltpu.async_copy(target_hbm.at[src], tbufs[slot], sems[slot])
        return (dp, dt)

    descs = [None] * NCHUNK
    descs[0] = start(0)
    zero = jnp.zeros((L,), jnp.int32)
    acc = (zero, zero, zero, zero)
    for c in range(NCHUNK):
        if c + 1 < NCHUNK:
            descs[c + 1] = start(c + 1)
        for d in descs[c]:
            d.wait()
        pb = pbufs[c % 2]
        tb = tbufs[c % 2]

        def body(i, carry, pb=pb, tb=tb):
            a0, a1, a2, a3 = carry
            accs = [a0, a1, a2, a3]
            row = pl.multiple_of(i * (GPR * L), GPR * L)
            for k in range(GPR):
                p = pb[pl.ds(row + k * L, L)]
                t = tb[pl.ds(row + k * L, L)]
                corr = (p < 1.25 * t) & (t < 1.25 * p)
                valid = t > 0.0
                step = jnp.where(corr, 0x10001, jnp.where(valid, 0x10000, 0))
                accs[k % 4] = accs[k % 4] + step
            return tuple(accs)

        acc = plsc.parallel_loop(0, ROWS, 1, carry=acc)(body)

    a0, a1, a2, a3 = acc
    obuf[...] = (a0 + a1) + (a2 + a3)
    pltpu.sync_copy(obuf, out_hbm.at[wid])


def kernel(pred, target):
    partials = _delta1_sc(pred.reshape(-1), target.reshape(-1))
    sum_c = jnp.sum(partials & 0xFFFF).astype(jnp.float32)
    sum_v = jnp.sum(partials >> 16).astype(jnp.float32)
    acc = sum_c / jnp.maximum(sum_v, 1.0)
    return jnp.where(sum_v < 10.0, jnp.float32(-1.0), acc)


# speedup vs baseline: 2.7835x; 2.0113x over previous
import functools

import jax
import jax.numpy as jnp
from jax import lax
from jax.experimental import pallas as pl
from jax.experimental.pallas import tpu as pltpu
from jax.experimental.pallas import tpu_sc as plsc

B = 32
H = 512
W = 512
NC = 2
NS = 16
L = 16
NW = NC * NS
ROWS_PER_CHUNK = 32
NCHUNK = H // ROWS_PER_CHUNK
GPW = W // L
GPR = 8

_mesh = plsc.VectorSubcoreMesh(core_axis_name="c", subcore_axis_name="s")


@functools.partial(
    pl.kernel,
    mesh=_mesh,
    out_type=jax.ShapeDtypeStruct((NW, L), jnp.int32),
    scratch_types=[
        pltpu.VMEM((ROWS_PER_CHUNK, W), jnp.float32),
        pltpu.VMEM((ROWS_PER_CHUNK, W), jnp.float32),
        pltpu.VMEM((ROWS_PER_CHUNK, W), jnp.float32),
        pltpu.VMEM((ROWS_PER_CHUNK, W), jnp.float32),
        pltpu.VMEM((L,), jnp.int32),
        pltpu.SemaphoreType.DMA,
        pltpu.SemaphoreType.DMA,
    ],
)
def _delta1_sc(pred_hbm, target_hbm, out_hbm, p0, p1, t0, t1, obuf, sem0, sem1):
    cid = lax.axis_index("c")
    sid = lax.axis_index("s")
    wid = sid * NC + cid
    pbufs = (p0, p1)
    tbufs = (t0, t1)
    sems = (sem0, sem1)

    def start(c, slot):
        rows = pl.ds(c * ROWS_PER_CHUNK, ROWS_PER_CHUNK)
        pltpu.async_copy(pred_hbm.at[wid, rows, :], pbufs[slot], sems[slot])
        pltpu.async_copy(target_hbm.at[wid, rows, :], tbufs[slot], sems[slot])

    def wait(slot):
        rows = pl.ds(0, ROWS_PER_CHUNK)
        pltpu.make_async_copy(pred_hbm.at[wid, rows, :], pbufs[slot], sems[slot]).wait()
        pltpu.make_async_copy(target_hbm.at[wid, rows, :], tbufs[slot], sems[slot]).wait()

    start(0, 0)
    start(1, 1)
    zero = jnp.zeros((L,), jnp.int32)

    def outer(j, acc):
        for slot in range(2):
            c = 2 * j + slot
            wait(slot)
            pb = pbufs[slot]
            tb = tbufs[slot]

            def body(i, carry, pb=pb, tb=tb):
                accs = list(carry)
                row = lax.shift_right_logical(i, 2)
                col = pl.multiple_of((i & 3) * (GPR * L), GPR * L)
                for k in range(GPR):
                    p = pb[row, pl.ds(col + k * L, L)]
                    t = tb[row, pl.ds(col + k * L, L)]
                    corr = (p < 1.25 * t) & (t < 1.25 * p)
                    valid = t > 0.0
                    step = jnp.where(corr, 0x10001, jnp.where(valid, 0x10000, 0))
                    accs[k % 4] = accs[k % 4] + step
                return tuple(accs)

            acc = plsc.parallel_loop(0, ROWS_PER_CHUNK * (GPW // GPR), 1, carry=acc)(body)

            @pl.when(c + 2 < NCHUNK)
            def _():
                start(c + 2, slot)

        return acc

    acc = lax.fori_loop(0, NCHUNK // 2, outer, (zero, zero, zero, zero))
    a0, a1, a2, a3 = acc
    obuf[...] = (a0 + a1) + (a2 + a3)
    pltpu.sync_copy(obuf, out_hbm.at[wid])


def kernel(pred, target):
    partials = _delta1_sc(pred, target)
    sum_c = jnp.sum(partials & 0xFFFF).astype(jnp.float32)
    sum_v = jnp.sum(partials >> 16).astype(jnp.float32)
    acc = sum_c / jnp.maximum(sum_v, 1.0)
    return jnp.where(sum_v < 10.0, jnp.float32(-1.0), acc)


# speedup vs baseline: 3.3760x; 1.2129x over previous
import functools

import jax
import jax.numpy as jnp
from jax import lax
from jax.experimental import pallas as pl
from jax.experimental.pallas import tpu as pltpu
from jax.experimental.pallas import tpu_sc as plsc

B = 32
H = 512
W = 512
NC = 2
NS = 16
L = 16
NW = NC * NS

SC_IMGS = 16
TC_IMGS = B - SC_IMGS
WPI = NW // SC_IMGS
KROWS = H // WPI

ROWS_PER_CHUNK = 32
NCHUNK = KROWS // ROWS_PER_CHUNK
GPW = W // L
GPR = 8

_mesh = plsc.VectorSubcoreMesh(core_axis_name="c", subcore_axis_name="s")


@functools.partial(
    pl.kernel,
    mesh=_mesh,
    out_type=jax.ShapeDtypeStruct((NW, L), jnp.int32),
    scratch_types=[
        pltpu.VMEM((ROWS_PER_CHUNK, W), jnp.float32),
        pltpu.VMEM((ROWS_PER_CHUNK, W), jnp.float32),
        pltpu.VMEM((ROWS_PER_CHUNK, W), jnp.float32),
        pltpu.VMEM((ROWS_PER_CHUNK, W), jnp.float32),
        pltpu.VMEM((L,), jnp.int32),
        pltpu.SemaphoreType.DMA,
        pltpu.SemaphoreType.DMA,
    ],
)
def _delta1_sc(pred_hbm, target_hbm, out_hbm, p0, p1, t0, t1, obuf, sem0, sem1):
    cid = lax.axis_index("c")
    sid = lax.axis_index("s")
    wid = sid * NC + cid
    img = lax.div(wid, WPI)
    r0 = lax.rem(wid, WPI) * KROWS
    pbufs = (p0, p1)
    tbufs = (t0, t1)
    sems = (sem0, sem1)

    def start(c, slot):
        rows = pl.ds(r0 + c * ROWS_PER_CHUNK, ROWS_PER_CHUNK)
        pltpu.async_copy(pred_hbm.at[img, rows, :], pbufs[slot], sems[slot])
        pltpu.async_copy(target_hbm.at[img, rows, :], tbufs[slot], sems[slot])

    def wait(slot):
        rows = pl.ds(0, ROWS_PER_CHUNK)
        pltpu.make_async_copy(pred_hbm.at[img, rows, :], pbufs[slot], sems[slot]).wait()
        pltpu.make_async_copy(target_hbm.at[img, rows, :], tbufs[slot], sems[slot]).wait()

    start(0, 0)
    start(1, 1)
    zero = jnp.zeros((L,), jnp.int32)

    def outer(j, acc):
        for slot in range(2):
            c = 2 * j + slot
            wait(slot)
            pb = pbufs[slot]
            tb = tbufs[slot]

            def body(i, carry, pb=pb, tb=tb):
                accs = list(carry)
                row = lax.shift_right_logical(i, 2)
                col = pl.multiple_of((i & 3) * (GPR * L), GPR * L)
                for k in range(GPR):
                    p = pb[row, pl.ds(col + k * L, L)]
                    t = tb[row, pl.ds(col + k * L, L)]
                    corr = (p < 1.25 * t) & (t < 1.25 * p)
                    valid = t > 0.0
                    step = jnp.where(corr, 0x10001, jnp.where(valid, 0x10000, 0))
                    accs[k % 4] = accs[k % 4] + step
                return tuple(accs)

            acc = plsc.parallel_loop(0, ROWS_PER_CHUNK * (GPW // GPR), 1, carry=acc)(body)

            @pl.when(c + 2 < NCHUNK)
            def _():
                start(c + 2, slot)

        return acc

    acc = lax.fori_loop(0, NCHUNK // 2, outer, (zero, zero, zero, zero))
    a0, a1, a2, a3 = acc
    obuf[...] = (a0 + a1) + (a2 + a3)
    pltpu.sync_copy(obuf, out_hbm.at[wid])


def _delta1_tc_body(p_ref, t_ref, o_ref):
    i = pl.program_id(0)

    @pl.when(i == 0)
    def _():
        o_ref[...] = jnp.zeros_like(o_ref)

    p = p_ref[0]
    t = t_ref[0]
    corr = (p < 1.25 * t) & (t < 1.25 * p)
    valid = t > 0.0
    csum = jnp.sum(corr.astype(jnp.float32))
    vsum = jnp.sum(valid.astype(jnp.float32))
    o_ref[0] = o_ref[0] + csum
    o_ref[1] = o_ref[1] + vsum


_delta1_tc = pl.pallas_call(
    _delta1_tc_body,
    grid=(TC_IMGS,),
    in_specs=[
        pl.BlockSpec((1, H, W), lambda i: (i + SC_IMGS, 0, 0)),
        pl.BlockSpec((1, H, W), lambda i: (i + SC_IMGS, 0, 0)),
    ],
    out_specs=pl.BlockSpec((2, 8, 128), lambda i: (0, 0, 0)),
    out_shape=jax.ShapeDtypeStruct((2, 8, 128), jnp.float32),
)


def kernel(pred, target):
    sc_partials = _delta1_sc(pred, target)
    tc_partials = _delta1_tc(pred, target)
    sum_c = jnp.sum(sc_partials & 0xFFFF).astype(jnp.float32) + tc_partials[0, 0, 0]
    sum_v = jnp.sum(sc_partials >> 16).astype(jnp.float32) + tc_partials[1, 0, 0]
    acc = sum_c / jnp.maximum(sum_v, 1.0)
    return jnp.where(sum_v < 10.0, jnp.float32(-1.0), acc)
